# baseline (speedup 1.0000x reference)
import functools

import jax
import jax.numpy as jnp
from jax import lax
from jax.experimental import pallas as pl
from jax.experimental.pallas import tpu as pltpu
from jax.experimental.pallas import tpu_sc as plsc

N = 10000
NP = 10240
E = 320000
DUMMY = N
NT = 16
EPT = 20480
EP = NT * EPT
RPT = NP // NT
TCB = 1024
WA = 144
ONE_COL = 128
LDIMS = [(128, 85), (85, 56), (56, 28), (28, 1)]


@functools.cache
def _make_sc_horner():
  mesh = plsc.VectorSubcoreMesh(core_axis_name="c", subcore_axis_name="s")

  @functools.partial(
      pl.kernel,
      mesh=mesh,
      compiler_params=pltpu.CompilerParams(use_tc_tiling_on_sc=False,
                                           needs_layout_passes=False),
      out_type=jax.ShapeDtypeStruct((NP,), jnp.float32),
      scratch_types=[
          pltpu.VMEM((NP,), jnp.float32),
          pltpu.VMEM((EPT,), jnp.int32),
          pltpu.VMEM((EPT,), jnp.int32),
          pltpu.VMEM((NP,), jnp.float32),
          pltpu.VMEM((NP,), jnp.float32),
          pltpu.VMEM((16, RPT), jnp.float32),
          pltpu.VMEM((RPT,), jnp.float32),
          pltpu.VMEM((RPT,), jnp.float32),
          pltpu.VMEM((4, RPT), jnp.float32),
          pltpu.VMEM_SHARED((16, NP), jnp.float32),
          pltpu.VMEM_SHARED((NP,), jnp.float32),
      ],
  )
  def sc_horner(y4_hbm, z3_hbm, z2_hbm, z1_hbm, z0_hbm, src_hbm, dst_hbm,
                out, pval_v, src_v, dst_v, acc_v, cnt_v, tmp_v, inv_v,
                ynew_v, z_v, sh_part, sh_y):
    c = lax.axis_index("c")
    s = lax.axis_index("s")
    r0 = s * RPT

    @pl.when(c == 0)
    def _body():
      pltpu.sync_copy(y4_hbm, pval_v)
      pltpu.sync_copy(src_hbm.at[s], src_v)
      pltpu.sync_copy(dst_hbm.at[s], dst_v)
      for j, z_hbm in enumerate((z3_hbm, z2_hbm, z1_hbm, z0_hbm)):
        pltpu.sync_copy(z_hbm.at[pl.ds(r0, RPT)], z_v.at[j])

      zeros16 = jnp.zeros((16,), jnp.float32)
      ones16 = jnp.ones((16,), jnp.float32)

      for step, k in enumerate((3, 2, 1, 0)):
        first = step == 0

        def zbody(i, carry):
          for u in range(4):
            acc_v[pl.ds(64 * i + 16 * u, 16)] = zeros16
          return carry

        lax.fori_loop(0, NP // 64, zbody, 0)
        if first:

          def czbody(i, carry):
            for u in range(4):
              cnt_v[pl.ds(64 * i + 16 * u, 16)] = zeros16
            return carry

          lax.fori_loop(0, NP // 64, czbody, 0)

        def ebody(e, carry):
          for u in range(4):
            o = 64 * e + 16 * u
            si = src_v[pl.ds(o, 16)]
            di = dst_v[pl.ds(o, 16)]
            vals = plsc.load_gather(pval_v, [si])
            plsc.addupdate_scatter(acc_v, [di], vals)
            if first:
              plsc.addupdate_scatter(cnt_v, [di], ones16)
          return carry

        lax.fori_loop(0, EPT // 64, ebody, 0)

        pltpu.sync_copy(acc_v, sh_part.at[s])
        plsc.subcore_barrier()
        pltpu.sync_copy(sh_part.at[:, pl.ds(r0, RPT)], tmp_v)

        if first:
          def sbody(i, carry):
            tot = tmp_v[0, pl.ds(16 * i, 16)]
            for t in range(1, 16):
              tot = tot + tmp_v[t, pl.ds(16 * i, 16)]
            ynew_v[pl.ds(16 * i, 16)] = tot
            return carry

          lax.fori_loop(0, RPT // 16, sbody, 0)
          plsc.subcore_barrier()
          pltpu.sync_copy(cnt_v, sh_part.at[s])
          plsc.subcore_barrier()
          pltpu.sync_copy(sh_part.at[:, pl.ds(r0, RPT)], tmp_v)

          def ibody(i, carry):
            tot = tmp_v[0, pl.ds(16 * i, 16)]
            for t in range(1, 16):
              tot = tot + tmp_v[t, pl.ds(16 * i, 16)]
            inv_v[pl.ds(16 * i, 16)] = 1.0 / jnp.maximum(tot, 1.0)
            ynew_v[pl.ds(16 * i, 16)] = (
                ynew_v[pl.ds(16 * i, 16)] * inv_v[pl.ds(16 * i, 16)]
                + z_v[0, pl.ds(16 * i, 16)])
            return carry

          lax.fori_loop(0, RPT // 16, ibody, 0)
        else:

          def rbody(i, carry):
            tot = tmp_v[0, pl.ds(16 * i, 16)]
            for t in range(1, 16):
              tot = tot + tmp_v[t, pl.ds(16 * i, 16)]
            ynew_v[pl.ds(16 * i, 16)] = (
                tot * inv_v[pl.ds(16 * i, 16)]
                + z_v[step, pl.ds(16 * i, 16)])
            return carry

          lax.fori_loop(0, RPT // 16, rbody, 0)

        if k > 0:
          pltpu.sync_copy(ynew_v, sh_y.at[pl.ds(r0, RPT)])
          plsc.subcore_barrier()
          pltpu.sync_copy(sh_y, pval_v)
        else:
          pltpu.sync_copy(ynew_v, out.at[pl.ds(r0, RPT)])

  return sc_horner


def _sc_horner_call(y4, z3, z2, z1, z0, src_t, dst_t):
  return _make_sc_horner()(y4, z3, z2, z1, z0, src_t, dst_t)


def _prep_body(eye_ref, b0, b1, b2, b3, c0, c1, c2, c3, mpack_ref):
  bs = [b0[...], b1[...], b2[...], b3[...]]
  cs = [c0[...], c1[...], c2[...], c3[...]]
  ms = [eye_ref[...], None, None, None, None]
  for l in range(4):
    new = []
    for k in range(5):
      t = None
      if ms[k] is not None:
        t = jnp.dot(ms[k], cs[l], preferred_element_type=jnp.float32)
      if k > 0 and ms[k - 1] is not None:
        tb = jnp.dot(ms[k - 1], bs[l], preferred_element_type=jnp.float32)
        t = tb if t is None else t + tb
      new.append(t)
    ms = new
  cols = [m[:, 0:1] for m in ms]
  cols.append(jnp.zeros((WA, 128 - 5), jnp.float32))
  mpack_ref[...] = jnp.concatenate(cols, axis=1)


def _z_body(x_ref, mp_ref, y4_ref, z3_ref, z2_ref, z1_ref, z0_ref):
  z = jnp.dot(x_ref[...], mp_ref[...], preferred_element_type=jnp.float32)
  y4_ref[...] = z[:, 4]
  z3_ref[...] = z[:, 3]
  z2_ref[...] = z[:, 2]
  z1_ref[...] = z[:, 1]
  z0_ref[...] = z[:, 0]


def _w_spec():
  return pl.BlockSpec((WA, WA), lambda i: (0, 0))


def _prep_call(eye, bs, cs):
  return pl.pallas_call(
      _prep_body,
      grid=(1,),
      in_specs=[_w_spec()] * 9,
      out_specs=pl.BlockSpec((WA, 128), lambda i: (0, 0)),
      out_shape=jax.ShapeDtypeStruct((WA, 128), jnp.float32),
  )(eye, *bs, *cs)


def _z_call(xp2, mpack):
  return pl.pallas_call(
      _z_body,
      grid=(NP // TCB,),
      in_specs=[
          pl.BlockSpec((TCB, WA), lambda i: (i, 0)),
          pl.BlockSpec((WA, 128), lambda i: (0, 0)),
      ],
      out_specs=[pl.BlockSpec((TCB,), lambda i: (i,))] * 5,
      out_shape=[jax.ShapeDtypeStruct((NP,), jnp.float32)] * 5,
  )(xp2, mpack)


def kernel(x, edge_index, Wl0, bl0, Wr0, Wl1, bl1, Wr1, Wl2, bl2, Wr2,
           Wl3, bl3, Wr3):
  f32 = jnp.float32
  ei = edge_index.astype(jnp.int32)
  pad_idx = jnp.full((EP - E,), DUMMY, jnp.int32)
  src_t = jnp.concatenate([ei[0], pad_idx]).reshape(NT, EPT)
  dst_t = jnp.concatenate([ei[1], pad_idx]).reshape(NT, EPT)

  xp2 = jnp.zeros((NP, WA), f32).at[:N, :128].set(x).at[:, ONE_COL].set(1.0)

  wls = [Wl0, Wl1, Wl2, Wl3]
  bls = [bl0, bl1, bl2, bl3]
  wrs = [Wr0, Wr1, Wr2, Wr3]
  bs, cs = [], []
  for l, (din, dout) in enumerate(LDIMS):
    bs.append(jnp.zeros((WA, WA), f32).at[:din, :dout].set(wls[l].T))
    cs.append(
        jnp.zeros((WA, WA), f32)
        .at[:din, :dout].set(wrs[l].T)
        .at[ONE_COL, :dout].set(bls[l])
        .at[ONE_COL, ONE_COL].set(1.0)
    )
  eye = jnp.eye(WA, dtype=f32)

  mpack = _prep_call(eye, bs, cs)
  y4, z3, z2, z1, z0 = _z_call(xp2, mpack)
  y = _sc_horner_call(y4, z3, z2, z1, z0, src_t, dst_t)
  return y[:N].reshape(N, 1)


# speedup vs baseline: 1.1931x; 1.1931x over previous
import functools

import jax
import jax.numpy as jnp
from jax import lax
from jax.experimental import pallas as pl
from jax.experimental.pallas import tpu as pltpu
from jax.experimental.pallas import tpu_sc as plsc

N = 10000
NP = 10240
E = 320000
DUMMY = N
NT = 16
EPT = 20480
EP = NT * EPT
RPT = NP // NT
TCB = 1024
WA = 144
ONE_COL = 128
LDIMS = [(128, 85), (85, 56), (56, 28), (28, 1)]


@functools.cache
def _make_sc_horner():
  mesh = plsc.VectorSubcoreMesh(core_axis_name="c", subcore_axis_name="s")

  @functools.partial(
      pl.kernel,
      mesh=mesh,
      compiler_params=pltpu.CompilerParams(use_tc_tiling_on_sc=False,
                                           needs_layout_passes=False),
      out_type=jax.ShapeDtypeStruct((NP,), jnp.float32),
      scratch_types=[
          pltpu.VMEM((NP,), jnp.float32),
          pltpu.VMEM((EPT,), jnp.int32),
          pltpu.VMEM((EPT,), jnp.int32),
          pltpu.VMEM((NP,), jnp.float32),
          pltpu.VMEM((NP,), jnp.float32),
          pltpu.VMEM((16, RPT), jnp.float32),
          pltpu.VMEM((RPT,), jnp.float32),
          pltpu.VMEM((RPT,), jnp.float32),
          pltpu.VMEM((4, RPT), jnp.float32),
          pltpu.VMEM_SHARED((16, NP), jnp.float32),
          pltpu.VMEM_SHARED((NP,), jnp.float32),
      ],
  )
  def sc_horner(y4_hbm, z3_hbm, z2_hbm, z1_hbm, z0_hbm, src_hbm, dst_hbm,
                out, pval_v, src_v, dst_v, acc_v, cnt_v, tmp_v, inv_v,
                ynew_v, z_v, sh_part, sh_y):
    c = lax.axis_index("c")
    s = lax.axis_index("s")
    r0 = s * RPT

    @pl.when(c == 0)
    def _body():
      pltpu.sync_copy(y4_hbm, pval_v)
      pltpu.sync_copy(src_hbm.at[s], src_v)
      pltpu.sync_copy(dst_hbm.at[s], dst_v)
      for j, z_hbm in enumerate((z3_hbm, z2_hbm, z1_hbm, z0_hbm)):
        pltpu.sync_copy(z_hbm.at[pl.ds(r0, RPT)], z_v.at[j])

      zeros16 = jnp.zeros((16,), jnp.float32)
      ones16 = jnp.ones((16,), jnp.float32)

      for step, k in enumerate((3, 2, 1, 0)):
        first = step == 0

        def zbody(i, carry):
          for u in range(4):
            acc_v[pl.ds(64 * i + 16 * u, 16)] = zeros16
          return carry

        lax.fori_loop(0, NP // 64, zbody, 0)
        if first:

          def czbody(i, carry):
            for u in range(4):
              cnt_v[pl.ds(64 * i + 16 * u, 16)] = zeros16
            return carry

          lax.fori_loop(0, NP // 64, czbody, 0)

        @plsc.parallel_loop(0, EPT // 16, unroll=4)
        def _edge_loop(e):
          o = 16 * e
          si = src_v[pl.ds(o, 16)]
          di = dst_v[pl.ds(o, 16)]
          vals = plsc.load_gather(pval_v, [si])
          plsc.addupdate_scatter(acc_v, [di], vals)
          if first:
            plsc.addupdate_scatter(cnt_v, [di], ones16)

        pltpu.sync_copy(acc_v, sh_part.at[s])
        plsc.subcore_barrier()
        pltpu.sync_copy(sh_part.at[:, pl.ds(r0, RPT)], tmp_v)

        if first:
          def sbody(i, carry):
            tot = tmp_v[0, pl.ds(16 * i, 16)]
            for t in range(1, 16):
              tot = tot + tmp_v[t, pl.ds(16 * i, 16)]
            ynew_v[pl.ds(16 * i, 16)] = tot
            return carry

          lax.fori_loop(0, RPT // 16, sbody, 0)
          plsc.subcore_barrier()
          pltpu.sync_copy(cnt_v, sh_part.at[s])
          plsc.subcore_barrier()
          pltpu.sync_copy(sh_part.at[:, pl.ds(r0, RPT)], tmp_v)

          def ibody(i, carry):
            tot = tmp_v[0, pl.ds(16 * i, 16)]
            for t in range(1, 16):
              tot = tot + tmp_v[t, pl.ds(16 * i, 16)]
            inv_v[pl.ds(16 * i, 16)] = 1.0 / jnp.maximum(tot, 1.0)
            ynew_v[pl.ds(16 * i, 16)] = (
                ynew_v[pl.ds(16 * i, 16)] * inv_v[pl.ds(16 * i, 16)]
                + z_v[0, pl.ds(16 * i, 16)])
            return carry

          lax.fori_loop(0, RPT // 16, ibody, 0)
        else:

          def rbody(i, carry):
            tot = tmp_v[0, pl.ds(16 * i, 16)]
            for t in range(1, 16):
              tot = tot + tmp_v[t, pl.ds(16 * i, 16)]
            ynew_v[pl.ds(16 * i, 16)] = (
                tot * inv_v[pl.ds(16 * i, 16)]
                + z_v[step, pl.ds(16 * i, 16)])
            return carry

          lax.fori_loop(0, RPT // 16, rbody, 0)

        if k > 0:
          pltpu.sync_copy(ynew_v, sh_y.at[pl.ds(r0, RPT)])
          plsc.subcore_barrier()
          pltpu.sync_copy(sh_y, pval_v)
        else:
          pltpu.sync_copy(ynew_v, out.at[pl.ds(r0, RPT)])

  return sc_horner


def _sc_horner_call(y4, z3, z2, z1, z0, src_t, dst_t):
  return _make_sc_horner()(y4, z3, z2, z1, z0, src_t, dst_t)


def _prep_body(eye_ref, b0, b1, b2, b3, c0, c1, c2, c3, mpack_ref):
  bs = [b0[...], b1[...], b2[...], b3[...]]
  cs = [c0[...], c1[...], c2[...], c3[...]]
  ms = [eye_ref[...], None, None, None, None]
  for l in range(4):
    new = []
    for k in range(5):
      t = None
      if ms[k] is not None:
        t = jnp.dot(ms[k], cs[l], preferred_element_type=jnp.float32)
      if k > 0 and ms[k - 1] is not None:
        tb = jnp.dot(ms[k - 1], bs[l], preferred_element_type=jnp.float32)
        t = tb if t is None else t + tb
      new.append(t)
    ms = new
  cols = [m[:, 0:1] for m in ms]
  cols.append(jnp.zeros((WA, 128 - 5), jnp.float32))
  mpack_ref[...] = jnp.concatenate(cols, axis=1)


def _z_body(x_ref, mp_ref, y4_ref, z3_ref, z2_ref, z1_ref, z0_ref):
  z = jnp.dot(x_ref[...], mp_ref[...], preferred_element_type=jnp.float32)
  y4_ref[...] = z[:, 4]
  z3_ref[...] = z[:, 3]
  z2_ref[...] = z[:, 2]
  z1_ref[...] = z[:, 1]
  z0_ref[...] = z[:, 0]


def _w_spec():
  return pl.BlockSpec((WA, WA), lambda i: (0, 0))


def _prep_call(eye, bs, cs):
  return pl.pallas_call(
      _prep_body,
      grid=(1,),
      in_specs=[_w_spec()] * 9,
      out_specs=pl.BlockSpec((WA, 128), lambda i: (0, 0)),
      out_shape=jax.ShapeDtypeStruct((WA, 128), jnp.float32),
  )(eye, *bs, *cs)


def _z_call(xp2, mpack):
  return pl.pallas_call(
      _z_body,
      grid=(NP // TCB,),
      in_specs=[
          pl.BlockSpec((TCB, WA), lambda i: (i, 0)),
          pl.BlockSpec((WA, 128), lambda i: (0, 0)),
      ],
      out_specs=[pl.BlockSpec((TCB,), lambda i: (i,))] * 5,
      out_shape=[jax.ShapeDtypeStruct((NP,), jnp.float32)] * 5,
  )(xp2, mpack)


def kernel(x, edge_index, Wl0, bl0, Wr0, Wl1, bl1, Wr1, Wl2, bl2, Wr2,
           Wl3, bl3, Wr3):
  f32 = jnp.float32
  ei = edge_index.astype(jnp.int32)
  pad_idx = jnp.full((EP - E,), DUMMY, jnp.int32)
  src_t = jnp.concatenate([ei[0], pad_idx]).reshape(NT, EPT)
  dst_t = jnp.concatenate([ei[1], pad_idx]).reshape(NT, EPT)

  xp2 = jnp.zeros((NP, WA), f32).at[:N, :128].set(x).at[:, ONE_COL].set(1.0)

  wls = [Wl0, Wl1, Wl2, Wl3]
  bls = [bl0, bl1, bl2, bl3]
  wrs = [Wr0, Wr1, Wr2, Wr3]
  bs, cs = [], []
  for l, (din, dout) in enumerate(LDIMS):
    bs.append(jnp.zeros((WA, WA), f32).at[:din, :dout].set(wls[l].T))
    cs.append(
        jnp.zeros((WA, WA), f32)
        .at[:din, :dout].set(wrs[l].T)
        .at[ONE_COL, :dout].set(bls[l])
        .at[ONE_COL, ONE_COL].set(1.0)
    )
  eye = jnp.eye(WA, dtype=f32)

  mpack = _prep_call(eye, bs, cs)
  y4, z3, z2, z1, z0 = _z_call(xp2, mpack)
  y = _sc_horner_call(y4, z3, z2, z1, z0, src_t, dst_t)
  return y[:N].reshape(N, 1)
